# baseline (speedup 1.0000x reference)
import jax
import jax.numpy as jnp
from jax import lax
from jax.experimental import pallas as pl
from jax.experimental.pallas import tpu as pltpu
from jax.experimental.pallas import tpu_sc as plsc

N = 50000
E = 800000
NC = 2
NS = 16
NR = 50048
RPT = NR // NS
K = 1000
K2 = 400
NPAD = 50176
SPT = NPAD // NS
RB = 1000
FH = 16

_MESH = plsc.VectorSubcoreMesh(core_axis_name="c", subcore_axis_name="s",
                               num_cores=NC, num_subcores=NS)
_SC_PARAMS = pltpu.CompilerParams(use_tc_tiling_on_sc=False)


def _zero_vec_rows(buf, nrows, width):
    def zrow(i, carry):
        for j0 in range(0, width, 16):
            buf[i, pl.ds(j0, 16)] = jnp.zeros((16,), jnp.float32)
        return carry
    lax.fori_loop(0, nrows, zrow, 0)


def _zero_vec_flat(buf, n):
    def zchunk(i, carry):
        buf[pl.ds(i * 16, 16)] = jnp.zeros((16,), jnp.float32)
        return carry
    lax.fori_loop(0, n // 16, zchunk, 0)


def _make_seg16(tmul, qoff):
    ept = E // NS
    nk = ept // K2
    ngroups = (nk + 2 + 3) // 4
    stage_sizes = [K2] * (RPT // K2) + ([RPT % K2] if RPT % K2 else [])

    def body(table, gidx, sidx, out, acc, slab, rows0, rows1, rows2, rows3,
             gi0, gi1, gi2, gi3, si0, si1, si2, si3,
             gs0, gs1, gs2, gs3, ss0, ss1, ss2, ss3):
        c = lax.axis_index("c")
        s = lax.axis_index("s")
        rows = [rows0, rows1, rows2, rows3]
        gi = [gi0, gi1, gi2, gi3]
        si = [si0, si1, si2, si3]
        gs = [gs0, gs1, gs2, gs3]
        ss = [ss0, ss1, ss2, ss3]
        rbase = s * RPT
        tb = (qoff + c) * NR + rbase
        off = 0
        for sz in stage_sizes:
            pltpu.sync_copy(table.at[pl.ds(tb + off, sz)],
                            rows0.at[pl.ds(0, sz)])
            pltpu.sync_copy(rows0.at[pl.ds(0, sz)],
                            slab.at[pl.ds(rbase + off, sz)])
            off += sz
        _zero_vec_rows(rows0, K2, FH)
        off = 0
        for sz in stage_sizes:
            pltpu.sync_copy(rows0.at[pl.ds(0, sz)],
                            acc.at[pl.ds(rbase + off, sz)])
            off += sz
        plsc.subcore_barrier()
        ebase = s * ept

        def group(g, carry):
            for slot in range(4):
                m = 4 * g + slot

                @pl.when(jnp.logical_and(m >= 4, m < nk + 4))
                def _():
                    pltpu.make_async_copy(
                        rows[slot], acc.at[si[slot]], ss[slot]).wait()

                @pl.when(m < nk)
                def _():
                    pltpu.sync_copy(gidx.at[pl.ds(ebase + m * K2, K2)],
                                    gi[slot])
                    pltpu.sync_copy(sidx.at[pl.ds(ebase + m * K2, K2)],
                                    si[slot])
                    pltpu.async_copy(slab.at[gi[slot]], rows[slot], gs[slot])

                mm = m - 2
                s2 = (slot + 2) % 4

                @pl.when(jnp.logical_and(mm >= 0, mm < nk))
                def _():
                    pltpu.make_async_copy(
                        slab.at[gi[s2]], rows[s2], gs[s2]).wait()
                    pltpu.async_copy(rows[s2], acc.at[si[s2]], ss[s2],
                                     add=True)
            return carry
        lax.fori_loop(0, ngroups, group, 0)
        for mm in range(4 * ngroups - 4, nk):
            slot = mm % 4
            pltpu.make_async_copy(rows[slot], acc.at[si[slot]],
                                  ss[slot]).wait()
        plsc.subcore_barrier()
        obase = c * NR + rbase
        off = 0
        for sz in stage_sizes:
            pltpu.sync_copy(acc.at[pl.ds(rbase + off, sz)],
                            rows0.at[pl.ds(0, sz)])
            pltpu.sync_copy(rows0.at[pl.ds(0, sz)],
                            out.at[pl.ds(obase + off, sz)])
            off += sz

    return pl.kernel(
        body,
        out_type=jax.ShapeDtypeStruct((2 * NR, FH), jnp.float32),
        mesh=_MESH,
        compiler_params=_SC_PARAMS,
        scratch_types=(
            [pltpu.VMEM_SHARED((NR, FH), jnp.float32),
             pltpu.VMEM_SHARED((NR, FH), jnp.float32)]
            + [pltpu.VMEM((K2, FH), jnp.float32) for _ in range(4)]
            + [pltpu.VMEM((K2,), jnp.int32) for _ in range(8)]
            + [pltpu.SemaphoreType.DMA for _ in range(8)]
        ),
    )


_seg16_2 = _make_seg16(2, 0)
_seg16_4a = _make_seg16(4, 0)
_seg16_4b = _make_seg16(4, 2)


def _deg_body(node, edge, dout, bout, accd, accb, zbuf, ones, ni, ei):
    c = lax.axis_index("c")
    s = lax.axis_index("s")
    ept = E // NC // NS
    _zero_vec_flat(zbuf, SPT)
    def orow(i, carry):
        ones[pl.ds(i * 16, 16)] = jnp.ones((16,), jnp.float32)
        return carry
    lax.fori_loop(0, 1008 // 16, orow, 0)
    sbase = s * SPT
    pltpu.sync_copy(zbuf, accd.at[pl.ds(sbase, SPT)])
    pltpu.sync_copy(zbuf, accb.at[pl.ds(sbase, SPT)])
    plsc.subcore_barrier()
    ebase = (c * NS + s) * ept

    def step(kk, carry):
        pltpu.sync_copy(node.at[pl.ds(ebase + kk * K, K)], ni)
        pltpu.sync_copy(edge.at[pl.ds(ebase + kk * K, K)], ei)
        pltpu.sync_copy(ones.at[pl.ds(0, K)], accd.at[ni], add=True)
        pltpu.sync_copy(ones.at[pl.ds(0, K)], accb.at[ei], add=True)
        return carry
    lax.fori_loop(0, ept // K, step, 0)
    plsc.subcore_barrier()
    pltpu.sync_copy(accd.at[pl.ds(sbase, SPT)], zbuf)
    pltpu.sync_copy(zbuf, dout.at[pl.ds(c * NPAD + sbase, SPT)])
    pltpu.sync_copy(accb.at[pl.ds(sbase, SPT)], zbuf)
    pltpu.sync_copy(zbuf, bout.at[pl.ds(c * NPAD + sbase, SPT)])


_seg_deg = pl.kernel(
    _deg_body,
    out_type=(jax.ShapeDtypeStruct((2 * NPAD,), jnp.float32),
              jax.ShapeDtypeStruct((2 * NPAD,), jnp.float32)),
    mesh=_MESH,
    compiler_params=_SC_PARAMS,
    scratch_types=[
        pltpu.VMEM_SHARED((NPAD,), jnp.float32),
        pltpu.VMEM_SHARED((NPAD,), jnp.float32),
        pltpu.VMEM((SPT,), jnp.float32),
        pltpu.VMEM((1008,), jnp.float32),
        pltpu.VMEM((K,), jnp.int32),
        pltpu.VMEM((K,), jnp.int32),
    ],
)


def _wsum_body(dinv, node, edge, out, acc, zbuf, ni, ei, rows, sem):
    c = lax.axis_index("c")
    s = lax.axis_index("s")
    ept = E // NC // NS
    _zero_vec_flat(zbuf, SPT)
    sbase = s * SPT
    pltpu.sync_copy(zbuf, acc.at[pl.ds(sbase, SPT)])
    plsc.subcore_barrier()
    ebase = (c * NS + s) * ept

    def step(kk, carry):
        pltpu.sync_copy(node.at[pl.ds(ebase + kk * K, K)], ni)
        pltpu.sync_copy(edge.at[pl.ds(ebase + kk * K, K)], ei)
        pltpu.async_copy(dinv.at[ni], rows, sem).wait()
        pltpu.sync_copy(rows, acc.at[ei], add=True)
        return carry
    lax.fori_loop(0, ept // K, step, 0)
    plsc.subcore_barrier()
    pltpu.sync_copy(acc.at[pl.ds(sbase, SPT)], zbuf)
    pltpu.sync_copy(zbuf, out.at[pl.ds(c * NPAD + sbase, SPT)])


_seg_wsum = pl.kernel(
    _wsum_body,
    out_type=jax.ShapeDtypeStruct((2 * NPAD,), jnp.float32),
    mesh=_MESH,
    compiler_params=_SC_PARAMS,
    scratch_types=[
        pltpu.VMEM_SHARED((NPAD,), jnp.float32),
        pltpu.VMEM((SPT,), jnp.float32),
        pltpu.VMEM((K,), jnp.int32),
        pltpu.VMEM((K,), jnp.int32),
        pltpu.VMEM((K,), jnp.float32),
        pltpu.SemaphoreType.DMA,
    ],
)


def _dotT(a, w):
    return lax.dot_general(a, w, (((1,), (1,)), ((), ())),
                           preferred_element_type=jnp.float32)


def _prep_body(xp, xl, w1, wt1, xw1, xe, mu1):
    v1 = _dotT(xp[...], w1[...])
    xw1[0, :, :] = v1[:, :FH]
    xw1[1, :, :] = v1[:, FH:]
    v = _dotT(xl[...], wt1[...])
    xe[...] = v
    m = jnp.maximum(v, 0.0)
    for q in range(4):
        mu1[q, :, :] = m[:, q * FH:(q + 1) * FH]


def _ef1_body(ep, bp, ef1, binv):
    b = bp[:, 0:1] + bp[:, 1:2]
    bi = jnp.where(b > 0, 1.0 / b, 0.0)
    binv[...] = bi
    ef1[0, :, :] = bi * ep[0]
    ef1[1, :, :] = bi * ep[1]


def _conv2_body(op, dp, b1, w2, xw2, dinv):
    d = dp[:, 0:1] + dp[:, 1:2]
    di = jnp.where(d > 0, 1.0 / d, 0.0)
    dinv[...] = di
    o = jnp.concatenate([op[0], op[1]], axis=1)
    z = jnp.maximum(di * o + b1[...], 0.0)
    v = _dotT(z, w2[...])
    for q in range(4):
        xw2[q, :, :] = v[:, q * FH:(q + 1) * FH]


def _hh_body(e01, e23, wp, binv, acc):
    i = pl.program_id(0)
    v = (wp[:, 0:1] + wp[:, 1:2]) * binv[...]
    ef2 = jnp.concatenate([e01[0], e01[1], e23[0], e23[1]], axis=1)
    part = lax.dot_general(v, ef2, (((0,), (0,)), ((), ())),
                           preferred_element_type=jnp.float32)
    @pl.when(i == 0)
    def _():
        acc[...] = jnp.zeros_like(acc)
    acc[...] += part


def _s2v_mu(xe, mi01, mi23, mo01, mo23, wt2, wt3):
    m_in = jnp.concatenate([mi01[0], mi01[1], mi23[0], mi23[1]], axis=1)
    m_out = jnp.concatenate([mo01[0], mo01[1], mo23[0], mo23[1]], axis=1)
    return jnp.maximum(
        xe[...] + _dotT(m_in, wt2[...]) + _dotT(m_out, wt3[...]), 0.0)


def _s2v_body(xe, mi01, mi23, mo01, mo23, wt2, wt3, mu):
    m = _s2v_mu(xe, mi01, mi23, mo01, mo23, wt2, wt3)
    for q in range(4):
        mu[q, :, :] = m[:, q * FH:(q + 1) * FH]


def _s2v_final_body(xe, mi01, mi23, mo01, mo23, wt2, wt3, acc):
    i = pl.program_id(0)
    m = _s2v_mu(xe, mi01, mi23, mo01, mo23, wt2, wt3)
    part = jnp.sum(m, axis=0, keepdims=True)
    @pl.when(i == 0)
    def _():
        acc[...] = jnp.zeros_like(acc)
    acc[...] += part


def _fuse_body(hh, hg, b2, wf, bf, out):
    h_h = hh[...] * (1.0 / N) + b2[...]
    ht = jnp.concatenate([h_h, hg[...]], axis=1)
    out[...] = jnp.maximum(_dotT(ht, wf[...]) + bf[...], 0.0)


_GRID = N // RB


def _rows(shape):
    return pl.BlockSpec(shape, lambda i: (i,) + (0,) * (len(shape) - 1))


def _slabs(shape):
    return pl.BlockSpec(shape, lambda i: (0, i, 0))


def _full(shape):
    return pl.BlockSpec(shape, lambda i: (0,) * len(shape))


def _f32(shape):
    return jax.ShapeDtypeStruct(shape, jnp.float32)


_prep = pl.pallas_call(
    _prep_body, grid=(_GRID,),
    in_specs=[_rows((RB, 4)), _rows((RB, 5)), _full((32, 4)), _full((64, 5))],
    out_specs=[_slabs((2, RB, FH)), _rows((RB, 64)), _slabs((4, RB, FH))],
    out_shape=[_f32((2, NR, FH)), _f32((N, 64)), _f32((4, NR, FH))],
)

_ef1 = pl.pallas_call(
    _ef1_body, grid=(_GRID,),
    in_specs=[_slabs((2, RB, FH)), _rows((RB, 2))],
    out_specs=[_slabs((2, RB, FH)), _rows((RB, 1))],
    out_shape=[_f32((2, NR, FH)), _f32((N, 1))],
)

_conv2 = pl.pallas_call(
    _conv2_body, grid=(_GRID,),
    in_specs=[_slabs((2, RB, FH)), _rows((RB, 2)), _full((1, 32)),
              _full((64, 32))],
    out_specs=[_slabs((4, RB, FH)), _rows((RB, 1))],
    out_shape=[_f32((4, NR, FH)), _f32((N, 1))],
)

_hh = pl.pallas_call(
    _hh_body, grid=(_GRID,),
    in_specs=[_slabs((2, RB, FH)), _slabs((2, RB, FH)), _rows((RB, 2)),
              _rows((RB, 1))],
    out_specs=_full((1, 64)),
    out_shape=_f32((1, 64)),
)

_s2v_specs = [_rows((RB, 64)), _slabs((2, RB, FH)), _slabs((2, RB, FH)),
              _slabs((2, RB, FH)), _slabs((2, RB, FH)),
              _full((64, 64)), _full((64, 64))]

_s2v = pl.pallas_call(
    _s2v_body, grid=(_GRID,),
    in_specs=_s2v_specs,
    out_specs=_slabs((4, RB, FH)),
    out_shape=_f32((4, NR, FH)),
)

_s2v_final = pl.pallas_call(
    _s2v_final_body, grid=(_GRID,),
    in_specs=_s2v_specs,
    out_specs=_full((1, 64)),
    out_shape=_f32((1, 64)),
)

_fuse = pl.pallas_call(
    _fuse_body, grid=(1,),
    in_specs=[_full((1, 64)), _full((1, 64)), _full((1, 64)),
              _full((128, 128)), _full((1, 128))],
    out_specs=_full((1, 128)),
    out_shape=_f32((1, 128)),
)


def kernel(x_phy, hyperedge_index, x_log, edge_index, W_hg1, b_hg1, W_hg2,
           b_hg2, W_t1, W_t2, W_t3, W_fuse, b_fuse):
    node, edge = hyperedge_index[0], hyperedge_index[1]
    src, dst = edge_index[0], edge_index[1]

    xw1, xe, mu = _prep(x_phy, x_log, W_hg1, W_t1)

    dp_flat, bp_flat = _seg_deg(node, edge)
    dp = dp_flat.reshape(2, NPAD).T
    bp = bp_flat.reshape(2, NPAD).T

    def _p2(flat_out):
        return flat_out.reshape(2, NR, FH)

    ef1_p = _p2(_seg16_2(xw1.reshape(2 * NR, FH), node, edge))
    ef1, binv = _ef1(ef1_p, bp)
    out1_p = _p2(_seg16_2(ef1.reshape(2 * NR, FH), edge, node))
    xw2, dinv = _conv2(out1_p, dp, b_hg1.reshape(1, 32), W_hg2)
    wsum_flat = _seg_wsum(dinv.reshape(N), node, edge)
    wp = wsum_flat.reshape(2, NPAD).T
    xw2_flat = xw2.reshape(4 * NR, FH)
    ef2_01 = _p2(_seg16_4a(xw2_flat, node, edge))
    ef2_23 = _p2(_seg16_4b(xw2_flat, node, edge))
    hh = _hh(ef2_01, ef2_23, wp, binv)

    for it in range(2):
        flat = mu.reshape(4 * NR, FH)
        mi01 = _p2(_seg16_4a(flat, src, dst))
        mi23 = _p2(_seg16_4b(flat, src, dst))
        mo01 = _p2(_seg16_4a(flat, dst, src))
        mo23 = _p2(_seg16_4b(flat, dst, src))
        if it == 0:
            mu = _s2v(xe, mi01, mi23, mo01, mo23, W_t2, W_t3)
        else:
            hg = _s2v_final(xe, mi01, mi23, mo01, mo23, W_t2, W_t3)

    out = _fuse(hh, hg, b_hg2.reshape(1, 64), W_fuse, b_fuse.reshape(1, 128))
    return out.reshape(128)


# speedup vs baseline: 1.2667x; 1.2667x over previous
import jax
import jax.numpy as jnp
from jax import lax
from jax.experimental import pallas as pl
from jax.experimental.pallas import tpu as pltpu
from jax.experimental.pallas import tpu_sc as plsc

N = 50000
E = 800000
NC = 2
NS = 16
NR = 50048
RPT = NR // NS
K = 1000
NPAD = 50176
SPT = NPAD // NS
RB = 1000
F = 32

_MESH = plsc.VectorSubcoreMesh(core_axis_name="c", subcore_axis_name="s",
                               num_cores=NC, num_subcores=NS)
_SC_PARAMS = pltpu.CompilerParams(use_tc_tiling_on_sc=False)


def _zero_vec_rows(buf, nrows, width):
    def zrow(i, carry):
        for j0 in range(0, width, 16):
            buf[i, pl.ds(j0, 16)] = jnp.zeros((16,), jnp.float32)
        return carry
    lax.fori_loop(0, nrows, zrow, 0)


def _zero_vec_flat(buf, n):
    def zchunk(i, carry):
        buf[pl.ds(i * 16, 16)] = jnp.zeros((16,), jnp.float32)
        return carry
    lax.fori_loop(0, n // 16, zchunk, 0)


def _make_seg32(edge_split):
    if edge_split:
        ept = E // NC // NS
        kc = 200
    else:
        ept = E // NS
        kc = 400
    nk = ept // kc
    npairs = (nk + 1) // 2
    stage_sizes = []
    off = 0
    while off < RPT:
        sz = min(2 * kc, RPT - off)
        stage_sizes.append(sz)
        off += sz

    def body(table, gidx, sidx, out, acc, rows_a, rows_b, gi_a, gi_b,
             si_a, si_b, sem_a, sem_b):
        c = lax.axis_index("c")
        s = lax.axis_index("s")
        rbase = s * RPT
        _zero_vec_rows(rows_a, kc, F)
        _zero_vec_rows(rows_b, kc, F)
        off = 0
        for sz in stage_sizes:
            if sz > kc:
                pltpu.sync_copy(rows_a, acc.at[pl.ds(rbase + off, kc)])
                pltpu.sync_copy(rows_b.at[pl.ds(0, sz - kc)],
                                acc.at[pl.ds(rbase + off + kc, sz - kc)])
            else:
                pltpu.sync_copy(rows_a.at[pl.ds(0, sz)],
                                acc.at[pl.ds(rbase + off, sz)])
            off += sz
        plsc.subcore_barrier()
        if edge_split:
            gbase = (c * NS + s) * ept
            sbase = gbase
        else:
            gbase = c * E + s * ept
            sbase = s * ept

        def load_issue(m, gi, si, rows, sem):
            pltpu.sync_copy(gidx.at[pl.ds(gbase + m * kc, kc)], gi)
            pltpu.sync_copy(sidx.at[pl.ds(sbase + m * kc, kc)], si)
            pltpu.async_copy(table.at[gi], rows, sem)

        def finish(gi, si, rows, sem):
            pltpu.make_async_copy(table.at[gi], rows, sem).wait()
            pltpu.sync_copy(rows, acc.at[si], add=True)

        load_issue(0, gi_a, si_a, rows_a, sem_a)

        def pair(j, carry):
            m1 = 2 * j + 1
            m2 = 2 * j + 2

            @pl.when(m1 < nk)
            def _():
                load_issue(m1, gi_b, si_b, rows_b, sem_b)
            finish(gi_a, si_a, rows_a, sem_a)

            @pl.when(m2 < nk)
            def _():
                load_issue(m2, gi_a, si_a, rows_a, sem_a)

            @pl.when(m1 < nk)
            def _():
                finish(gi_b, si_b, rows_b, sem_b)
            return carry
        lax.fori_loop(0, npairs, pair, 0)
        plsc.subcore_barrier()
        obase = c * NR + rbase
        off = 0
        for sz in stage_sizes:
            if sz > kc:
                pltpu.sync_copy(acc.at[pl.ds(rbase + off, kc)], rows_a)
                pltpu.sync_copy(acc.at[pl.ds(rbase + off + kc, sz - kc)],
                                rows_b.at[pl.ds(0, sz - kc)])
                pltpu.sync_copy(rows_a, out.at[pl.ds(obase + off, kc)])
                pltpu.sync_copy(rows_b.at[pl.ds(0, sz - kc)],
                                out.at[pl.ds(obase + off + kc, sz - kc)])
            else:
                pltpu.sync_copy(acc.at[pl.ds(rbase + off, sz)],
                                rows_a.at[pl.ds(0, sz)])
                pltpu.sync_copy(rows_a.at[pl.ds(0, sz)],
                                out.at[pl.ds(obase + off, sz)])
            off += sz

    return pl.kernel(
        body,
        out_type=jax.ShapeDtypeStruct((2 * NR, F), jnp.float32),
        mesh=_MESH,
        compiler_params=_SC_PARAMS,
        scratch_types=(
            [pltpu.VMEM_SHARED((NR, F), jnp.float32)]
            + [pltpu.VMEM((kc, F), jnp.float32) for _ in range(2)]
            + [pltpu.VMEM((kc,), jnp.int32) for _ in range(4)]
            + [pltpu.SemaphoreType.DMA for _ in range(2)]
        ),
    )


_seg32_edge = _make_seg32(True)
_seg32_feat = _make_seg32(False)


def _deg_body(node, edge, dout, bout, accd, accb, zbuf, ones, ni, ei):
    c = lax.axis_index("c")
    s = lax.axis_index("s")
    ept = E // NC // NS
    _zero_vec_flat(zbuf, SPT)
    def orow(i, carry):
        ones[pl.ds(i * 16, 16)] = jnp.ones((16,), jnp.float32)
        return carry
    lax.fori_loop(0, 1008 // 16, orow, 0)
    sbase = s * SPT
    pltpu.sync_copy(zbuf, accd.at[pl.ds(sbase, SPT)])
    pltpu.sync_copy(zbuf, accb.at[pl.ds(sbase, SPT)])
    plsc.subcore_barrier()
    ebase = (c * NS + s) * ept

    def step(kk, carry):
        pltpu.sync_copy(node.at[pl.ds(ebase + kk * K, K)], ni)
        pltpu.sync_copy(edge.at[pl.ds(ebase + kk * K, K)], ei)
        pltpu.sync_copy(ones.at[pl.ds(0, K)], accd.at[ni], add=True)
        pltpu.sync_copy(ones.at[pl.ds(0, K)], accb.at[ei], add=True)
        return carry
    lax.fori_loop(0, ept // K, step, 0)
    plsc.subcore_barrier()
    pltpu.sync_copy(accd.at[pl.ds(sbase, SPT)], zbuf)
    pltpu.sync_copy(zbuf, dout.at[pl.ds(c * NPAD + sbase, SPT)])
    pltpu.sync_copy(accb.at[pl.ds(sbase, SPT)], zbuf)
    pltpu.sync_copy(zbuf, bout.at[pl.ds(c * NPAD + sbase, SPT)])


_seg_deg = pl.kernel(
    _deg_body,
    out_type=(jax.ShapeDtypeStruct((2 * NPAD,), jnp.float32),
              jax.ShapeDtypeStruct((2 * NPAD,), jnp.float32)),
    mesh=_MESH,
    compiler_params=_SC_PARAMS,
    scratch_types=[
        pltpu.VMEM_SHARED((NPAD,), jnp.float32),
        pltpu.VMEM_SHARED((NPAD,), jnp.float32),
        pltpu.VMEM((SPT,), jnp.float32),
        pltpu.VMEM((1008,), jnp.float32),
        pltpu.VMEM((K,), jnp.int32),
        pltpu.VMEM((K,), jnp.int32),
    ],
)


def _wsum_body(dinv, node, edge, out, acc, zbuf, ni, ei, rows, sem):
    c = lax.axis_index("c")
    s = lax.axis_index("s")
    ept = E // NC // NS
    _zero_vec_flat(zbuf, SPT)
    sbase = s * SPT
    pltpu.sync_copy(zbuf, acc.at[pl.ds(sbase, SPT)])
    plsc.subcore_barrier()
    ebase = (c * NS + s) * ept

    def step(kk, carry):
        pltpu.sync_copy(node.at[pl.ds(ebase + kk * K, K)], ni)
        pltpu.sync_copy(edge.at[pl.ds(ebase + kk * K, K)], ei)
        pltpu.async_copy(dinv.at[ni], rows, sem).wait()
        pltpu.sync_copy(rows, acc.at[ei], add=True)
        return carry
    lax.fori_loop(0, ept // K, step, 0)
    plsc.subcore_barrier()
    pltpu.sync_copy(acc.at[pl.ds(sbase, SPT)], zbuf)
    pltpu.sync_copy(zbuf, out.at[pl.ds(c * NPAD + sbase, SPT)])


_seg_wsum = pl.kernel(
    _wsum_body,
    out_type=jax.ShapeDtypeStruct((2 * NPAD,), jnp.float32),
    mesh=_MESH,
    compiler_params=_SC_PARAMS,
    scratch_types=[
        pltpu.VMEM_SHARED((NPAD,), jnp.float32),
        pltpu.VMEM((SPT,), jnp.float32),
        pltpu.VMEM((K,), jnp.int32),
        pltpu.VMEM((K,), jnp.int32),
        pltpu.VMEM((K,), jnp.float32),
        pltpu.SemaphoreType.DMA,
    ],
)


def _dotT(a, w):
    return lax.dot_general(a, w, (((1,), (1,)), ((), ())),
                           preferred_element_type=jnp.float32)


def _prep_body(xp, xl, w1, wt1, xw1, xe, mu1):
    xw1[...] = _dotT(xp[...], w1[...])
    v = _dotT(xl[...], wt1[...])
    xe[...] = v
    m = jnp.maximum(v, 0.0)
    mu1[0, :, :] = m[:, :F]
    mu1[1, :, :] = m[:, F:]


def _ef1_body(ep, bp, ef1, binv):
    b = bp[:, 0:1] + bp[:, 1:2]
    bi = jnp.where(b > 0, 1.0 / b, 0.0)
    binv[...] = bi
    ef1[...] = bi * (ep[0] + ep[1])


def _conv2_body(op, dp, b1, w2, xw2, dinv):
    d = dp[:, 0:1] + dp[:, 1:2]
    di = jnp.where(d > 0, 1.0 / d, 0.0)
    dinv[...] = di
    z = jnp.maximum(di * (op[0] + op[1]) + b1[...], 0.0)
    v = _dotT(z, w2[...])
    xw2[0, :, :] = v[:, :F]
    xw2[1, :, :] = v[:, F:]


def _hh_body(ep, wp, binv, acc):
    i = pl.program_id(0)
    v = (wp[:, 0:1] + wp[:, 1:2]) * binv[...]
    ef2 = jnp.concatenate([ep[0], ep[1]], axis=1)
    part = lax.dot_general(v, ef2, (((0,), (0,)), ((), ())),
                           preferred_element_type=jnp.float32)
    @pl.when(i == 0)
    def _():
        acc[...] = jnp.zeros_like(acc)
    acc[...] += part


def _s2v_mu(xe, mi, mo, wt2, wt3):
    m_in = jnp.concatenate([mi[0], mi[1]], axis=1)
    m_out = jnp.concatenate([mo[0], mo[1]], axis=1)
    return jnp.maximum(
        xe[...] + _dotT(m_in, wt2[...]) + _dotT(m_out, wt3[...]), 0.0)


def _s2v_body(xe, mi, mo, wt2, wt3, mu):
    m = _s2v_mu(xe, mi, mo, wt2, wt3)
    mu[0, :, :] = m[:, :F]
    mu[1, :, :] = m[:, F:]


def _s2v_final_body(xe, mi, mo, wt2, wt3, acc):
    i = pl.program_id(0)
    m = _s2v_mu(xe, mi, mo, wt2, wt3)
    part = jnp.sum(m, axis=0, keepdims=True)
    @pl.when(i == 0)
    def _():
        acc[...] = jnp.zeros_like(acc)
    acc[...] += part


def _fuse_body(hh, hg, b2, wf, bf, out):
    h_h = hh[...] * (1.0 / N) + b2[...]
    ht = jnp.concatenate([h_h, hg[...]], axis=1)
    out[...] = jnp.maximum(_dotT(ht, wf[...]) + bf[...], 0.0)


_GRID = N // RB


def _rows(shape):
    return pl.BlockSpec(shape, lambda i: (i,) + (0,) * (len(shape) - 1))


def _slabs(shape):
    return pl.BlockSpec(shape, lambda i: (0, i, 0))


def _full(shape):
    return pl.BlockSpec(shape, lambda i: (0,) * len(shape))


def _f32(shape):
    return jax.ShapeDtypeStruct(shape, jnp.float32)


_prep = pl.pallas_call(
    _prep_body, grid=(_GRID,),
    in_specs=[_rows((RB, 4)), _rows((RB, 5)), _full((32, 4)), _full((64, 5))],
    out_specs=[_rows((RB, F)), _rows((RB, 64)), _slabs((2, RB, F))],
    out_shape=[_f32((NR, F)), _f32((N, 64)), _f32((2, NR, F))],
)

_ef1 = pl.pallas_call(
    _ef1_body, grid=(_GRID,),
    in_specs=[_slabs((2, RB, F)), _rows((RB, 2))],
    out_specs=[_rows((RB, F)), _rows((RB, 1))],
    out_shape=[_f32((NR, F)), _f32((N, 1))],
)

_conv2 = pl.pallas_call(
    _conv2_body, grid=(_GRID,),
    in_specs=[_slabs((2, RB, F)), _rows((RB, 2)), _full((1, 32)),
              _full((64, 32))],
    out_specs=[_slabs((2, RB, F)), _rows((RB, 1))],
    out_shape=[_f32((2, NR, F)), _f32((N, 1))],
)

_hh = pl.pallas_call(
    _hh_body, grid=(_GRID,),
    in_specs=[_slabs((2, RB, F)), _rows((RB, 2)), _rows((RB, 1))],
    out_specs=_full((1, 64)),
    out_shape=_f32((1, 64)),
)

_s2v_specs = [_rows((RB, 64)), _slabs((2, RB, F)), _slabs((2, RB, F)),
              _full((64, 64)), _full((64, 64))]

_s2v = pl.pallas_call(
    _s2v_body, grid=(_GRID,),
    in_specs=_s2v_specs,
    out_specs=_slabs((2, RB, F)),
    out_shape=_f32((2, NR, F)),
)

_s2v_final = pl.pallas_call(
    _s2v_final_body, grid=(_GRID,),
    in_specs=_s2v_specs,
    out_specs=_full((1, 64)),
    out_shape=_f32((1, 64)),
)

_fuse = pl.pallas_call(
    _fuse_body, grid=(1,),
    in_specs=[_full((1, 64)), _full((1, 64)), _full((1, 64)),
              _full((128, 128)), _full((1, 128))],
    out_specs=_full((1, 128)),
    out_shape=_f32((1, 128)),
)


def kernel(x_phy, hyperedge_index, x_log, edge_index, W_hg1, b_hg1, W_hg2,
           b_hg2, W_t1, W_t2, W_t3, W_fuse, b_fuse):
    node, edge = hyperedge_index[0], hyperedge_index[1]
    src, dst = edge_index[0], edge_index[1]
    node2 = jnp.concatenate([node, node + NR])
    src2 = jnp.concatenate([src, src + NR])
    dst2 = jnp.concatenate([dst, dst + NR])

    xw1, xe, mu = _prep(x_phy, x_log, W_hg1, W_t1)

    dp_flat, bp_flat = _seg_deg(node, edge)
    dp = dp_flat.reshape(2, NPAD).T
    bp = bp_flat.reshape(2, NPAD).T

    def _p2(flat_out):
        return flat_out.reshape(2, NR, F)

    ef1_p = _p2(_seg32_edge(xw1, node, edge))
    ef1, binv = _ef1(ef1_p, bp)
    out1_p = _p2(_seg32_edge(ef1, edge, node))
    xw2, dinv = _conv2(out1_p, dp, b_hg1.reshape(1, 32), W_hg2)
    wsum_flat = _seg_wsum(dinv.reshape(N), node, edge)
    wp = wsum_flat.reshape(2, NPAD).T
    ef2_p = _p2(_seg32_feat(xw2.reshape(2 * NR, F), node2, edge))
    hh = _hh(ef2_p, wp, binv)

    for it in range(2):
        flat = mu.reshape(2 * NR, F)
        mi_p = _p2(_seg32_feat(flat, src2, dst))
        mo_p = _p2(_seg32_feat(flat, dst2, src))
        if it == 0:
            mu = _s2v(xe, mi_p, mo_p, W_t2, W_t3)
        else:
            hg = _s2v_final(xe, mi_p, mo_p, W_t2, W_t3)

    out = _fuse(hh, hg, b_hg2.reshape(1, 64), W_fuse, b_fuse.reshape(1, 128))
    return out.reshape(128)


# speedup vs baseline: 1.3165x; 1.0393x over previous
import jax
import jax.numpy as jnp
from jax import lax
from jax.experimental import pallas as pl
from jax.experimental.pallas import tpu as pltpu
from jax.experimental.pallas import tpu_sc as plsc

N = 50000
E = 800000
NC = 2
NS = 16
NR = 50048
RPT = NR // NS
K = 1000
K2 = 1000
NPAD = 50176
SPT = NPAD // NS
RB = 1000
FH = 16

_MESH = plsc.VectorSubcoreMesh(core_axis_name="c", subcore_axis_name="s",
                               num_cores=NC, num_subcores=NS)
_SC_PARAMS = pltpu.CompilerParams(use_tc_tiling_on_sc=False)


def _zero_vec_rows(buf, nrows, width):
    def zrow(i, carry):
        for j0 in range(0, width, 16):
            buf[i, pl.ds(j0, 16)] = jnp.zeros((16,), jnp.float32)
        return carry
    lax.fori_loop(0, nrows, zrow, 0)


def _zero_vec_flat(buf, n):
    def zchunk(i, carry):
        buf[pl.ds(i * 16, 16)] = jnp.zeros((16,), jnp.float32)
        return carry
    lax.fori_loop(0, n // 16, zchunk, 0)


_EPT = E // NS
_NK = _EPT // K2
_NGROUPS = (_NK + 2 + 3) // 4
_STAGE = [K2] * (RPT // K2) + ([RPT % K2] if RPT % K2 else [])


def _make_seg16(n_dirs, n_qoff):
    def body(*refs):
        i = 0
        table = refs[i]; i += 1
        gidxs = refs[i:i + n_dirs]; i += n_dirs
        sidxs = refs[i:i + n_dirs]; i += n_dirs
        outs = refs[i:i + n_dirs]; i += n_dirs
        acc = refs[i]; i += 1
        rows = refs[i:i + 4]; i += 4
        gi = refs[i:i + 4]; i += 4
        si = refs[i:i + 4]; i += 4
        gs = refs[i:i + 4]; i += 4
        ss = refs[i:i + 4]; i += 4
        c = lax.axis_index("c")
        s = lax.axis_index("s")
        rbase = s * RPT

        for d in range(n_dirs):
            for q in range(n_qoff):
                qoff = 2 * q
                gidx, sidx, out = gidxs[d], sidxs[d], outs[d]
                _zero_vec_rows(rows[0], K2, FH)
                off = 0
                for sz in _STAGE:
                    pltpu.sync_copy(rows[0].at[pl.ds(0, sz)],
                                    acc.at[pl.ds(rbase + off, sz)])
                    off += sz
                plsc.subcore_barrier()
                gbase = (qoff + c) * E + s * _EPT
                sbase = s * _EPT

                def group(g, carry):
                    for slot in range(4):
                        m = 4 * g + slot

                        @pl.when(jnp.logical_and(m >= 4, m < _NK + 4))
                        def _():
                            pltpu.make_async_copy(
                                rows[slot], acc.at[si[slot]],
                                ss[slot]).wait()

                        @pl.when(m < _NK)
                        def _():
                            pltpu.sync_copy(
                                gidx.at[pl.ds(gbase + m * K2, K2)], gi[slot])
                            pltpu.sync_copy(
                                sidx.at[pl.ds(sbase + m * K2, K2)], si[slot])
                            pltpu.async_copy(table.at[gi[slot]], rows[slot],
                                             gs[slot])

                        mm = m - 2
                        s2 = (slot + 2) % 4

                        @pl.when(jnp.logical_and(mm >= 0, mm < _NK))
                        def _():
                            pltpu.make_async_copy(
                                table.at[gi[s2]], rows[s2], gs[s2]).wait()
                            pltpu.async_copy(rows[s2], acc.at[si[s2]],
                                             ss[s2], add=True)
                    return carry
                lax.fori_loop(0, _NGROUPS, group, 0)
                for mm in range(4 * _NGROUPS - 4, _NK):
                    slot = mm % 4
                    pltpu.make_async_copy(rows[slot], acc.at[si[slot]],
                                          ss[slot]).wait()
                plsc.subcore_barrier()
                obase = (qoff + c) * NR + rbase
                off = 0
                for sz in _STAGE:
                    pltpu.sync_copy(acc.at[pl.ds(rbase + off, sz)],
                                    rows[0].at[pl.ds(0, sz)])
                    pltpu.sync_copy(rows[0].at[pl.ds(0, sz)],
                                    out.at[pl.ds(obase + off, sz)])
                    off += sz

    out_sd = jax.ShapeDtypeStruct((2 * n_qoff * NR, FH), jnp.float32)
    return pl.kernel(
        body,
        out_type=tuple([out_sd] * n_dirs) if n_dirs > 1 else out_sd,
        mesh=_MESH,
        compiler_params=_SC_PARAMS,
        scratch_types=(
            [pltpu.VMEM_SHARED((NR, FH), jnp.float32)]
            + [pltpu.VMEM((K2, FH), jnp.float32) for _ in range(4)]
            + [pltpu.VMEM((K2,), jnp.int32) for _ in range(8)]
            + [pltpu.SemaphoreType.DMA for _ in range(8)]
        ),
    )


_seg16_2 = _make_seg16(n_dirs=1, n_qoff=1)
_seg16_4 = _make_seg16(n_dirs=1, n_qoff=2)
_seg16_msg = _make_seg16(n_dirs=2, n_qoff=2)


def _deg_body(node, edge, dout, bout, accd, accb, zbuf, ones, ni, ei):
    c = lax.axis_index("c")
    s = lax.axis_index("s")
    ept = E // NC // NS
    _zero_vec_flat(zbuf, SPT)
    def orow(i, carry):
        ones[pl.ds(i * 16, 16)] = jnp.ones((16,), jnp.float32)
        return carry
    lax.fori_loop(0, 1008 // 16, orow, 0)
    sbase = s * SPT
    pltpu.sync_copy(zbuf, accd.at[pl.ds(sbase, SPT)])
    pltpu.sync_copy(zbuf, accb.at[pl.ds(sbase, SPT)])
    plsc.subcore_barrier()
    ebase = (c * NS + s) * ept

    def step(kk, carry):
        pltpu.sync_copy(node.at[pl.ds(ebase + kk * K, K)], ni)
        pltpu.sync_copy(edge.at[pl.ds(ebase + kk * K, K)], ei)
        pltpu.sync_copy(ones.at[pl.ds(0, K)], accd.at[ni], add=True)
        pltpu.sync_copy(ones.at[pl.ds(0, K)], accb.at[ei], add=True)
        return carry
    lax.fori_loop(0, ept // K, step, 0)
    plsc.subcore_barrier()
    pltpu.sync_copy(accd.at[pl.ds(sbase, SPT)], zbuf)
    pltpu.sync_copy(zbuf, dout.at[pl.ds(c * NPAD + sbase, SPT)])
    pltpu.sync_copy(accb.at[pl.ds(sbase, SPT)], zbuf)
    pltpu.sync_copy(zbuf, bout.at[pl.ds(c * NPAD + sbase, SPT)])


_seg_deg = pl.kernel(
    _deg_body,
    out_type=(jax.ShapeDtypeStruct((2 * NPAD,), jnp.float32),
              jax.ShapeDtypeStruct((2 * NPAD,), jnp.float32)),
    mesh=_MESH,
    compiler_params=_SC_PARAMS,
    scratch_types=[
        pltpu.VMEM_SHARED((NPAD,), jnp.float32),
        pltpu.VMEM_SHARED((NPAD,), jnp.float32),
        pltpu.VMEM((SPT,), jnp.float32),
        pltpu.VMEM((1008,), jnp.float32),
        pltpu.VMEM((K,), jnp.int32),
        pltpu.VMEM((K,), jnp.int32),
    ],
)


def _wsum_body(dinv, node, edge, out, acc, zbuf, ni, ei, rows, sem):
    c = lax.axis_index("c")
    s = lax.axis_index("s")
    ept = E // NC // NS
    _zero_vec_flat(zbuf, SPT)
    sbase = s * SPT
    pltpu.sync_copy(zbuf, acc.at[pl.ds(sbase, SPT)])
    plsc.subcore_barrier()
    ebase = (c * NS + s) * ept

    def step(kk, carry):
        pltpu.sync_copy(node.at[pl.ds(ebase + kk * K, K)], ni)
        pltpu.sync_copy(edge.at[pl.ds(ebase + kk * K, K)], ei)
        pltpu.async_copy(dinv.at[ni], rows, sem).wait()
        pltpu.sync_copy(rows, acc.at[ei], add=True)
        return carry
    lax.fori_loop(0, ept // K, step, 0)
    plsc.subcore_barrier()
    pltpu.sync_copy(acc.at[pl.ds(sbase, SPT)], zbuf)
    pltpu.sync_copy(zbuf, out.at[pl.ds(c * NPAD + sbase, SPT)])


_seg_wsum = pl.kernel(
    _wsum_body,
    out_type=jax.ShapeDtypeStruct((2 * NPAD,), jnp.float32),
    mesh=_MESH,
    compiler_params=_SC_PARAMS,
    scratch_types=[
        pltpu.VMEM_SHARED((NPAD,), jnp.float32),
        pltpu.VMEM((SPT,), jnp.float32),
        pltpu.VMEM((K,), jnp.int32),
        pltpu.VMEM((K,), jnp.int32),
        pltpu.VMEM((K,), jnp.float32),
        pltpu.SemaphoreType.DMA,
    ],
)


def _dotT(a, w):
    return lax.dot_general(a, w, (((1,), (1,)), ((), ())),
                           preferred_element_type=jnp.float32)


def _prep_body(xp, xl, w1, wt1, xw1, xe, mu1):
    v1 = _dotT(xp[...], w1[...])
    xw1[0, :, :] = v1[:, :FH]
    xw1[1, :, :] = v1[:, FH:]
    v = _dotT(xl[...], wt1[...])
    xe[...] = v
    m = jnp.maximum(v, 0.0)
    for q in range(4):
        mu1[q, :, :] = m[:, q * FH:(q + 1) * FH]


def _ef1_body(ep, bp, ef1, binv):
    b = bp[:, 0:1] + bp[:, 1:2]
    bi = jnp.where(b > 0, 1.0 / b, 0.0)
    binv[...] = bi
    ef1[0, :, :] = bi * ep[0]
    ef1[1, :, :] = bi * ep[1]


def _conv2_body(op, dp, b1, w2, xw2, dinv):
    d = dp[:, 0:1] + dp[:, 1:2]
    di = jnp.where(d > 0, 1.0 / d, 0.0)
    dinv[...] = di
    o = jnp.concatenate([op[0], op[1]], axis=1)
    z = jnp.maximum(di * o + b1[...], 0.0)
    v = _dotT(z, w2[...])
    for q in range(4):
        xw2[q, :, :] = v[:, q * FH:(q + 1) * FH]


def _hh_body(eq, wp, binv, acc):
    i = pl.program_id(0)
    v = (wp[:, 0:1] + wp[:, 1:2]) * binv[...]
    ef2 = jnp.concatenate([eq[0], eq[1], eq[2], eq[3]], axis=1)
    part = lax.dot_general(v, ef2, (((0,), (0,)), ((), ())),
                           preferred_element_type=jnp.float32)
    @pl.when(i == 0)
    def _():
        acc[...] = jnp.zeros_like(acc)
    acc[...] += part


def _s2v_mu(xe, mi, mo, wt2, wt3):
    m_in = jnp.concatenate([mi[0], mi[1], mi[2], mi[3]], axis=1)
    m_out = jnp.concatenate([mo[0], mo[1], mo[2], mo[3]], axis=1)
    return jnp.maximum(
        xe[...] + _dotT(m_in, wt2[...]) + _dotT(m_out, wt3[...]), 0.0)


def _s2v_body(xe, mi, mo, wt2, wt3, mu):
    m = _s2v_mu(xe, mi, mo, wt2, wt3)
    for q in range(4):
        mu[q, :, :] = m[:, q * FH:(q + 1) * FH]


def _s2v_final_body(xe, mi, mo, wt2, wt3, acc):
    i = pl.program_id(0)
    m = _s2v_mu(xe, mi, mo, wt2, wt3)
    part = jnp.sum(m, axis=0, keepdims=True)
    @pl.when(i == 0)
    def _():
        acc[...] = jnp.zeros_like(acc)
    acc[...] += part


def _fuse_body(hh, hg, b2, wf, bf, out):
    h_h = hh[...] * (1.0 / N) + b2[...]
    ht = jnp.concatenate([h_h, hg[...]], axis=1)
    out[...] = jnp.maximum(_dotT(ht, wf[...]) + bf[...], 0.0)


_GRID = N // RB


def _rows(shape):
    return pl.BlockSpec(shape, lambda i: (i,) + (0,) * (len(shape) - 1))


def _slabs(shape):
    return pl.BlockSpec(shape, lambda i: (0, i, 0))


def _full(shape):
    return pl.BlockSpec(shape, lambda i: (0,) * len(shape))


def _f32(shape):
    return jax.ShapeDtypeStruct(shape, jnp.float32)


_prep = pl.pallas_call(
    _prep_body, grid=(_GRID,),
    in_specs=[_rows((RB, 4)), _rows((RB, 5)), _full((32, 4)), _full((64, 5))],
    out_specs=[_slabs((2, RB, FH)), _rows((RB, 64)), _slabs((4, RB, FH))],
    out_shape=[_f32((2, NR, FH)), _f32((N, 64)), _f32((4, NR, FH))],
)

_ef1 = pl.pallas_call(
    _ef1_body, grid=(_GRID,),
    in_specs=[_slabs((2, RB, FH)), _rows((RB, 2))],
    out_specs=[_slabs((2, RB, FH)), _rows((RB, 1))],
    out_shape=[_f32((2, NR, FH)), _f32((N, 1))],
)

_conv2 = pl.pallas_call(
    _conv2_body, grid=(_GRID,),
    in_specs=[_slabs((2, RB, FH)), _rows((RB, 2)), _full((1, 32)),
              _full((64, 32))],
    out_specs=[_slabs((4, RB, FH)), _rows((RB, 1))],
    out_shape=[_f32((4, NR, FH)), _f32((N, 1))],
)

_hh = pl.pallas_call(
    _hh_body, grid=(_GRID,),
    in_specs=[_slabs((4, RB, FH)), _rows((RB, 2)), _rows((RB, 1))],
    out_specs=_full((1, 64)),
    out_shape=_f32((1, 64)),
)

_s2v_specs = [_rows((RB, 64)), _slabs((4, RB, FH)), _slabs((4, RB, FH)),
              _full((64, 64)), _full((64, 64))]

_s2v = pl.pallas_call(
    _s2v_body, grid=(_GRID,),
    in_specs=_s2v_specs,
    out_specs=_slabs((4, RB, FH)),
    out_shape=_f32((4, NR, FH)),
)

_s2v_final = pl.pallas_call(
    _s2v_final_body, grid=(_GRID,),
    in_specs=_s2v_specs,
    out_specs=_full((1, 64)),
    out_shape=_f32((1, 64)),
)

_fuse = pl.pallas_call(
    _fuse_body, grid=(1,),
    in_specs=[_full((1, 64)), _full((1, 64)), _full((1, 64)),
              _full((128, 128)), _full((1, 128))],
    out_specs=_full((1, 128)),
    out_shape=_f32((1, 128)),
)


def kernel(x_phy, hyperedge_index, x_log, edge_index, W_hg1, b_hg1, W_hg2,
           b_hg2, W_t1, W_t2, W_t3, W_fuse, b_fuse):
    node, edge = hyperedge_index[0], hyperedge_index[1]
    src, dst = edge_index[0], edge_index[1]
    node2 = jnp.concatenate([node, node + NR])
    edge2 = jnp.concatenate([edge, edge + NR])
    node4 = jnp.concatenate([node, node + NR, node + 2 * NR, node + 3 * NR])
    src4 = jnp.concatenate([src, src + NR, src + 2 * NR, src + 3 * NR])
    dst4 = jnp.concatenate([dst, dst + NR, dst + 2 * NR, dst + 3 * NR])

    xw1, xe, mu = _prep(x_phy, x_log, W_hg1, W_t1)

    dp_flat, bp_flat = _seg_deg(node, edge)
    dp = dp_flat.reshape(2, NPAD).T
    bp = bp_flat.reshape(2, NPAD).T

    ef1_p = _seg16_2(xw1.reshape(2 * NR, FH), node2, edge).reshape(2, NR, FH)
    ef1, binv = _ef1(ef1_p, bp)
    out1_p = _seg16_2(ef1.reshape(2 * NR, FH), edge2, node).reshape(2, NR, FH)
    xw2, dinv = _conv2(out1_p, dp, b_hg1.reshape(1, 32), W_hg2)
    wsum_flat = _seg_wsum(dinv.reshape(N), node, edge)
    wp = wsum_flat.reshape(2, NPAD).T
    ef2_p = _seg16_4(xw2.reshape(4 * NR, FH), node4, edge).reshape(4, NR, FH)
    hh = _hh(ef2_p, wp, binv)

    for it in range(2):
        flat = mu.reshape(4 * NR, FH)
        mi_f, mo_f = _seg16_msg(flat, src4, dst4, dst, src)
        mi_p = mi_f.reshape(4, NR, FH)
        mo_p = mo_f.reshape(4, NR, FH)
        if it == 0:
            mu = _s2v(xe, mi_p, mo_p, W_t2, W_t3)
        else:
            hg = _s2v_final(xe, mi_p, mo_p, W_t2, W_t3)

    out = _fuse(hh, hg, b_hg2.reshape(1, 64), W_fuse, b_fuse.reshape(1, 128))
    return out.reshape(128)


# speedup vs baseline: 1.3452x; 1.0218x over previous
import jax
import jax.numpy as jnp
from jax import lax
from jax.experimental import pallas as pl
from jax.experimental.pallas import tpu as pltpu
from jax.experimental.pallas import tpu_sc as plsc

N = 50000
E = 800000
NC = 2
NS = 16
NR = 50048
RPT = NR // NS
K = 5000
K2 = 1000
NPAD = 50176
SPT = NPAD // NS
RB = 1000
FH = 16

_MESH = plsc.VectorSubcoreMesh(core_axis_name="c", subcore_axis_name="s",
                               num_cores=NC, num_subcores=NS)
_SC_PARAMS = pltpu.CompilerParams(use_tc_tiling_on_sc=False)


def _zero_vec_rows(buf, nrows, width):
    def zrow(i, carry):
        for j0 in range(0, width, 16):
            buf[i, pl.ds(j0, 16)] = jnp.zeros((16,), jnp.float32)
        return carry
    lax.fori_loop(0, nrows, zrow, 0)


def _zero_vec_flat(buf, n):
    def zchunk(i, carry):
        buf[pl.ds(i * 16, 16)] = jnp.zeros((16,), jnp.float32)
        return carry
    lax.fori_loop(0, n // 16, zchunk, 0)


_EPT = E // NS
_NK = _EPT // K2
_NGROUPS = (_NK + 2 + 3) // 4
_STAGE = [K2] * (RPT // K2) + ([RPT % K2] if RPT % K2 else [])


def _make_seg16(n_dirs, n_qoff):
    def body(*refs):
        i = 0
        table = refs[i]; i += 1
        gidxs = refs[i:i + n_dirs]; i += n_dirs
        sidxs = refs[i:i + n_dirs]; i += n_dirs
        outs = refs[i:i + n_dirs]; i += n_dirs
        acc = refs[i]; i += 1
        rows = refs[i:i + 4]; i += 4
        gi = refs[i:i + 4]; i += 4
        si = refs[i:i + 4]; i += 4
        gs = refs[i:i + 4]; i += 4
        ss = refs[i:i + 4]; i += 4
        c = lax.axis_index("c")
        s = lax.axis_index("s")
        rbase = s * RPT

        for d in range(n_dirs):
            for q in range(n_qoff):
                qoff = 2 * q
                gidx, sidx, out = gidxs[d], sidxs[d], outs[d]
                _zero_vec_rows(rows[0], K2, FH)
                off = 0
                for sz in _STAGE:
                    pltpu.sync_copy(rows[0].at[pl.ds(0, sz)],
                                    acc.at[pl.ds(rbase + off, sz)])
                    off += sz
                plsc.subcore_barrier()
                gbase = (qoff + c) * E + s * _EPT
                sbase = s * _EPT

                def group(g, carry):
                    for slot in range(4):
                        m = 4 * g + slot

                        @pl.when(jnp.logical_and(m >= 4, m < _NK + 4))
                        def _():
                            pltpu.make_async_copy(
                                rows[slot], acc.at[si[slot]],
                                ss[slot]).wait()

                        @pl.when(m < _NK)
                        def _():
                            pltpu.sync_copy(
                                gidx.at[pl.ds(gbase + m * K2, K2)], gi[slot])
                            pltpu.sync_copy(
                                sidx.at[pl.ds(sbase + m * K2, K2)], si[slot])
                            pltpu.async_copy(table.at[gi[slot]], rows[slot],
                                             gs[slot])

                        mm = m - 2
                        s2 = (slot + 2) % 4

                        @pl.when(jnp.logical_and(mm >= 0, mm < _NK))
                        def _():
                            pltpu.make_async_copy(
                                table.at[gi[s2]], rows[s2], gs[s2]).wait()
                            pltpu.async_copy(rows[s2], acc.at[si[s2]],
                                             ss[s2], add=True)
                    return carry
                lax.fori_loop(0, _NGROUPS, group, 0)
                for mm in range(4 * _NGROUPS - 4, _NK):
                    slot = mm % 4
                    pltpu.make_async_copy(rows[slot], acc.at[si[slot]],
                                          ss[slot]).wait()
                plsc.subcore_barrier()
                obase = (qoff + c) * NR + rbase
                off = 0
                for sz in _STAGE:
                    pltpu.sync_copy(acc.at[pl.ds(rbase + off, sz)],
                                    rows[0].at[pl.ds(0, sz)])
                    pltpu.sync_copy(rows[0].at[pl.ds(0, sz)],
                                    out.at[pl.ds(obase + off, sz)])
                    off += sz

    out_sd = jax.ShapeDtypeStruct((2 * n_qoff * NR, FH), jnp.float32)
    return pl.kernel(
        body,
        out_type=tuple([out_sd] * n_dirs) if n_dirs > 1 else out_sd,
        mesh=_MESH,
        compiler_params=_SC_PARAMS,
        scratch_types=(
            [pltpu.VMEM_SHARED((NR, FH), jnp.float32)]
            + [pltpu.VMEM((K2, FH), jnp.float32) for _ in range(4)]
            + [pltpu.VMEM((K2,), jnp.int32) for _ in range(8)]
            + [pltpu.SemaphoreType.DMA for _ in range(8)]
        ),
    )


_seg16_2 = _make_seg16(n_dirs=1, n_qoff=1)
_seg16_4 = _make_seg16(n_dirs=1, n_qoff=2)
_seg16_msg = _make_seg16(n_dirs=2, n_qoff=2)


def _deg_body(node, edge, dout, bout, accd, accb, zbuf, ones, ni, ei):
    c = lax.axis_index("c")
    s = lax.axis_index("s")
    ept = E // NC // NS
    _zero_vec_flat(zbuf, SPT)
    def orow(i, carry):
        ones[pl.ds(i * 16, 16)] = jnp.ones((16,), jnp.float32)
        return carry
    lax.fori_loop(0, (K + 16) // 16, orow, 0)
    sbase = s * SPT
    pltpu.sync_copy(zbuf, accd.at[pl.ds(sbase, SPT)])
    pltpu.sync_copy(zbuf, accb.at[pl.ds(sbase, SPT)])
    plsc.subcore_barrier()
    ebase = (c * NS + s) * ept

    def step(kk, carry):
        pltpu.sync_copy(node.at[pl.ds(ebase + kk * K, K)], ni)
        pltpu.sync_copy(edge.at[pl.ds(ebase + kk * K, K)], ei)
        pltpu.sync_copy(ones.at[pl.ds(0, K)], accd.at[ni], add=True)
        pltpu.sync_copy(ones.at[pl.ds(0, K)], accb.at[ei], add=True)
        return carry
    lax.fori_loop(0, ept // K, step, 0)
    plsc.subcore_barrier()
    pltpu.sync_copy(accd.at[pl.ds(sbase, SPT)], zbuf)
    pltpu.sync_copy(zbuf, dout.at[pl.ds(c * NPAD + sbase, SPT)])
    pltpu.sync_copy(accb.at[pl.ds(sbase, SPT)], zbuf)
    pltpu.sync_copy(zbuf, bout.at[pl.ds(c * NPAD + sbase, SPT)])


_seg_deg = pl.kernel(
    _deg_body,
    out_type=(jax.ShapeDtypeStruct((2 * NPAD,), jnp.float32),
              jax.ShapeDtypeStruct((2 * NPAD,), jnp.float32)),
    mesh=_MESH,
    compiler_params=_SC_PARAMS,
    scratch_types=[
        pltpu.VMEM_SHARED((NPAD,), jnp.float32),
        pltpu.VMEM_SHARED((NPAD,), jnp.float32),
        pltpu.VMEM((SPT,), jnp.float32),
        pltpu.VMEM((K + 16,), jnp.float32),
        pltpu.VMEM((K,), jnp.int32),
        pltpu.VMEM((K,), jnp.int32),
    ],
)


def _wsum_body(dinv, node, edge, out, acc, zbuf, ni, ei, rows, sem):
    c = lax.axis_index("c")
    s = lax.axis_index("s")
    ept = E // NC // NS
    _zero_vec_flat(zbuf, SPT)
    sbase = s * SPT
    pltpu.sync_copy(zbuf, acc.at[pl.ds(sbase, SPT)])
    plsc.subcore_barrier()
    ebase = (c * NS + s) * ept

    def step(kk, carry):
        pltpu.sync_copy(node.at[pl.ds(ebase + kk * K, K)], ni)
        pltpu.sync_copy(edge.at[pl.ds(ebase + kk * K, K)], ei)
        pltpu.async_copy(dinv.at[ni], rows, sem).wait()
        pltpu.sync_copy(rows, acc.at[ei], add=True)
        return carry
    lax.fori_loop(0, ept // K, step, 0)
    plsc.subcore_barrier()
    pltpu.sync_copy(acc.at[pl.ds(sbase, SPT)], zbuf)
    pltpu.sync_copy(zbuf, out.at[pl.ds(c * NPAD + sbase, SPT)])


_seg_wsum = pl.kernel(
    _wsum_body,
    out_type=jax.ShapeDtypeStruct((2 * NPAD,), jnp.float32),
    mesh=_MESH,
    compiler_params=_SC_PARAMS,
    scratch_types=[
        pltpu.VMEM_SHARED((NPAD,), jnp.float32),
        pltpu.VMEM((SPT,), jnp.float32),
        pltpu.VMEM((K,), jnp.int32),
        pltpu.VMEM((K,), jnp.int32),
        pltpu.VMEM((K,), jnp.float32),
        pltpu.SemaphoreType.DMA,
    ],
)


def _dotT(a, w):
    return lax.dot_general(a, w, (((1,), (1,)), ((), ())),
                           preferred_element_type=jnp.float32)


def _prep_body(xp, xl, w1, wt1, xw1, xe, mu1):
    v1 = _dotT(xp[...], w1[...])
    xw1[0, :, :] = v1[:, :FH]
    xw1[1, :, :] = v1[:, FH:]
    v = _dotT(xl[...], wt1[...])
    xe[...] = v
    m = jnp.maximum(v, 0.0)
    for q in range(4):
        mu1[q, :, :] = m[:, q * FH:(q + 1) * FH]


def _ef1_body(ep, bp, ef1, binv):
    b = bp[:, 0:1] + bp[:, 1:2]
    bi = jnp.where(b > 0, 1.0 / b, 0.0)
    binv[...] = bi
    ef1[0, :, :] = bi * ep[0]
    ef1[1, :, :] = bi * ep[1]


def _conv2_body(op, dp, b1, w2, xw2, dinv):
    d = dp[:, 0:1] + dp[:, 1:2]
    di = jnp.where(d > 0, 1.0 / d, 0.0)
    dinv[...] = di
    o = jnp.concatenate([op[0], op[1]], axis=1)
    z = jnp.maximum(di * o + b1[...], 0.0)
    v = _dotT(z, w2[...])
    for q in range(4):
        xw2[q, :, :] = v[:, q * FH:(q + 1) * FH]


def _hh_body(eq, wp, binv, acc):
    i = pl.program_id(0)
    v = (wp[:, 0:1] + wp[:, 1:2]) * binv[...]
    ef2 = jnp.concatenate([eq[0], eq[1], eq[2], eq[3]], axis=1)
    part = lax.dot_general(v, ef2, (((0,), (0,)), ((), ())),
                           preferred_element_type=jnp.float32)
    @pl.when(i == 0)
    def _():
        acc[...] = jnp.zeros_like(acc)
    acc[...] += part


def _s2v_mu(xe, mi, mo, wt2, wt3):
    m_in = jnp.concatenate([mi[0], mi[1], mi[2], mi[3]], axis=1)
    m_out = jnp.concatenate([mo[0], mo[1], mo[2], mo[3]], axis=1)
    return jnp.maximum(
        xe[...] + _dotT(m_in, wt2[...]) + _dotT(m_out, wt3[...]), 0.0)


def _s2v_body(xe, mi, mo, wt2, wt3, mu):
    m = _s2v_mu(xe, mi, mo, wt2, wt3)
    for q in range(4):
        mu[q, :, :] = m[:, q * FH:(q + 1) * FH]


def _s2v_final_body(xe, mi, mo, wt2, wt3, acc):
    i = pl.program_id(0)
    m = _s2v_mu(xe, mi, mo, wt2, wt3)
    part = jnp.sum(m, axis=0, keepdims=True)
    @pl.when(i == 0)
    def _():
        acc[...] = jnp.zeros_like(acc)
    acc[...] += part


def _fuse_body(hh, hg, b2, wf, bf, out):
    h_h = hh[...] * (1.0 / N) + b2[...]
    ht = jnp.concatenate([h_h, hg[...]], axis=1)
    out[...] = jnp.maximum(_dotT(ht, wf[...]) + bf[...], 0.0)


_GRID = N // RB


def _rows(shape):
    return pl.BlockSpec(shape, lambda i: (i,) + (0,) * (len(shape) - 1))


def _slabs(shape):
    return pl.BlockSpec(shape, lambda i: (0, i, 0))


def _full(shape):
    return pl.BlockSpec(shape, lambda i: (0,) * len(shape))


def _f32(shape):
    return jax.ShapeDtypeStruct(shape, jnp.float32)


_prep = pl.pallas_call(
    _prep_body, grid=(_GRID,),
    in_specs=[_rows((RB, 4)), _rows((RB, 5)), _full((32, 4)), _full((64, 5))],
    out_specs=[_slabs((2, RB, FH)), _rows((RB, 64)), _slabs((4, RB, FH))],
    out_shape=[_f32((2, NR, FH)), _f32((N, 64)), _f32((4, NR, FH))],
)

_ef1 = pl.pallas_call(
    _ef1_body, grid=(_GRID,),
    in_specs=[_slabs((2, RB, FH)), _rows((RB, 2))],
    out_specs=[_slabs((2, RB, FH)), _rows((RB, 1))],
    out_shape=[_f32((2, NR, FH)), _f32((N, 1))],
)

_conv2 = pl.pallas_call(
    _conv2_body, grid=(_GRID,),
    in_specs=[_slabs((2, RB, FH)), _rows((RB, 2)), _full((1, 32)),
              _full((64, 32))],
    out_specs=[_slabs((4, RB, FH)), _rows((RB, 1))],
    out_shape=[_f32((4, NR, FH)), _f32((N, 1))],
)

_hh = pl.pallas_call(
    _hh_body, grid=(_GRID,),
    in_specs=[_slabs((4, RB, FH)), _rows((RB, 2)), _rows((RB, 1))],
    out_specs=_full((1, 64)),
    out_shape=_f32((1, 64)),
)

_s2v_specs = [_rows((RB, 64)), _slabs((4, RB, FH)), _slabs((4, RB, FH)),
              _full((64, 64)), _full((64, 64))]

_s2v = pl.pallas_call(
    _s2v_body, grid=(_GRID,),
    in_specs=_s2v_specs,
    out_specs=_slabs((4, RB, FH)),
    out_shape=_f32((4, NR, FH)),
)

_s2v_final = pl.pallas_call(
    _s2v_final_body, grid=(_GRID,),
    in_specs=_s2v_specs,
    out_specs=_full((1, 64)),
    out_shape=_f32((1, 64)),
)

_fuse = pl.pallas_call(
    _fuse_body, grid=(1,),
    in_specs=[_full((1, 64)), _full((1, 64)), _full((1, 64)),
              _full((128, 128)), _full((1, 128))],
    out_specs=_full((1, 128)),
    out_shape=_f32((1, 128)),
)


def kernel(x_phy, hyperedge_index, x_log, edge_index, W_hg1, b_hg1, W_hg2,
           b_hg2, W_t1, W_t2, W_t3, W_fuse, b_fuse):
    node, edge = hyperedge_index[0], hyperedge_index[1]
    src, dst = edge_index[0], edge_index[1]
    node2 = jnp.concatenate([node, node + NR])
    edge2 = jnp.concatenate([edge, edge + NR])
    node4 = jnp.concatenate([node, node + NR, node + 2 * NR, node + 3 * NR])
    src4 = jnp.concatenate([src, src + NR, src + 2 * NR, src + 3 * NR])
    dst4 = jnp.concatenate([dst, dst + NR, dst + 2 * NR, dst + 3 * NR])

    xw1, xe, mu = _prep(x_phy, x_log, W_hg1, W_t1)

    dp_flat, bp_flat = _seg_deg(node, edge)
    dp = dp_flat.reshape(2, NPAD).T
    bp = bp_flat.reshape(2, NPAD).T

    ef1_p = _seg16_2(xw1.reshape(2 * NR, FH), node2, edge).reshape(2, NR, FH)
    ef1, binv = _ef1(ef1_p, bp)
    out1_p = _seg16_2(ef1.reshape(2 * NR, FH), edge2, node).reshape(2, NR, FH)
    xw2, dinv = _conv2(out1_p, dp, b_hg1.reshape(1, 32), W_hg2)
    wsum_flat = _seg_wsum(dinv.reshape(N), node, edge)
    wp = wsum_flat.reshape(2, NPAD).T
    ef2_p = _seg16_4(xw2.reshape(4 * NR, FH), node4, edge).reshape(4, NR, FH)
    hh = _hh(ef2_p, wp, binv)

    for it in range(2):
        flat = mu.reshape(4 * NR, FH)
        mi_p = _seg16_4(flat, src4, dst).reshape(4, NR, FH)
        mo_p = _seg16_4(flat, dst4, src).reshape(4, NR, FH)
        if it == 0:
            mu = _s2v(xe, mi_p, mo_p, W_t2, W_t3)
        else:
            hg = _s2v_final(xe, mi_p, mo_p, W_t2, W_t3)

    out = _fuse(hh, hg, b_hg2.reshape(1, 64), W_fuse, b_fuse.reshape(1, 128))
    return out.reshape(128)


# speedup vs baseline: 1.3517x; 1.0049x over previous
import jax
import jax.numpy as jnp
from jax import lax
from jax.experimental import pallas as pl
from jax.experimental.pallas import tpu as pltpu
from jax.experimental.pallas import tpu_sc as plsc

N = 50000
E = 800000
NC = 2
NS = 16
NR = 50048
RPT = NR // NS
K = 25000
K2 = 1000
NPAD = 50176
SPT = NPAD // NS
RB = 1000
FH = 16

_MESH = plsc.VectorSubcoreMesh(core_axis_name="c", subcore_axis_name="s",
                               num_cores=NC, num_subcores=NS)
_SC_PARAMS = pltpu.CompilerParams(use_tc_tiling_on_sc=False)


def _zero_vec_rows(buf, nrows, width):
    def zrow(i, carry):
        for j0 in range(0, width, 16):
            buf[i, pl.ds(j0, 16)] = jnp.zeros((16,), jnp.float32)
        return carry
    lax.fori_loop(0, nrows, zrow, 0)


def _zero_vec_flat(buf, n):
    def zchunk(i, carry):
        buf[pl.ds(i * 16, 16)] = jnp.zeros((16,), jnp.float32)
        return carry
    lax.fori_loop(0, n // 16, zchunk, 0)


_EPT = E // NS
_NK = _EPT // K2
_NGROUPS = (_NK + 2 + 3) // 4
_STAGE = [K2] * (RPT // K2) + ([RPT % K2] if RPT % K2 else [])


def _make_seg16(n_dirs, n_qoff):
    def body(*refs):
        i = 0
        table = refs[i]; i += 1
        gidxs = refs[i:i + n_dirs]; i += n_dirs
        sidxs = refs[i:i + n_dirs]; i += n_dirs
        outs = refs[i:i + n_dirs]; i += n_dirs
        acc = refs[i]; i += 1
        rows = refs[i:i + 4]; i += 4
        gi = refs[i:i + 4]; i += 4
        si = refs[i:i + 4]; i += 4
        gs = refs[i:i + 4]; i += 4
        ss = refs[i:i + 4]; i += 4
        c = lax.axis_index("c")
        s = lax.axis_index("s")
        rbase = s * RPT

        for d in range(n_dirs):
            for q in range(n_qoff):
                qoff = 2 * q
                gidx, sidx, out = gidxs[d], sidxs[d], outs[d]
                _zero_vec_rows(rows[0], K2, FH)
                off = 0
                for sz in _STAGE:
                    pltpu.sync_copy(rows[0].at[pl.ds(0, sz)],
                                    acc.at[pl.ds(rbase + off, sz)])
                    off += sz
                plsc.subcore_barrier()
                gbase = (qoff + c) * E + s * _EPT
                sbase = s * _EPT

                def group(g, carry):
                    for slot in range(4):
                        m = 4 * g + slot

                        @pl.when(jnp.logical_and(m >= 4, m < _NK + 4))
                        def _():
                            pltpu.make_async_copy(
                                rows[slot], acc.at[si[slot]],
                                ss[slot]).wait()

                        @pl.when(m < _NK)
                        def _():
                            pltpu.sync_copy(
                                gidx.at[pl.ds(gbase + m * K2, K2)], gi[slot])
                            pltpu.sync_copy(
                                sidx.at[pl.ds(sbase + m * K2, K2)], si[slot])
                            pltpu.async_copy(table.at[gi[slot]], rows[slot],
                                             gs[slot])

                        mm = m - 2
                        s2 = (slot + 2) % 4

                        @pl.when(jnp.logical_and(mm >= 0, mm < _NK))
                        def _():
                            pltpu.make_async_copy(
                                table.at[gi[s2]], rows[s2], gs[s2]).wait()
                            pltpu.async_copy(rows[s2], acc.at[si[s2]],
                                             ss[s2], add=True)
                    return carry
                lax.fori_loop(0, _NGROUPS, group, 0)
                for mm in range(4 * _NGROUPS - 4, _NK):
                    slot = mm % 4
                    pltpu.make_async_copy(rows[slot], acc.at[si[slot]],
                                          ss[slot]).wait()
                plsc.subcore_barrier()
                obase = (qoff + c) * NR + rbase
                off = 0
                for sz in _STAGE:
                    pltpu.sync_copy(acc.at[pl.ds(rbase + off, sz)],
                                    rows[0].at[pl.ds(0, sz)])
                    pltpu.sync_copy(rows[0].at[pl.ds(0, sz)],
                                    out.at[pl.ds(obase + off, sz)])
                    off += sz

    out_sd = jax.ShapeDtypeStruct((2 * n_qoff * NR, FH), jnp.float32)
    return pl.kernel(
        body,
        out_type=tuple([out_sd] * n_dirs) if n_dirs > 1 else out_sd,
        mesh=_MESH,
        compiler_params=_SC_PARAMS,
        scratch_types=(
            [pltpu.VMEM_SHARED((NR, FH), jnp.float32)]
            + [pltpu.VMEM((K2, FH), jnp.float32) for _ in range(4)]
            + [pltpu.VMEM((K2,), jnp.int32) for _ in range(8)]
            + [pltpu.SemaphoreType.DMA for _ in range(8)]
        ),
    )


_seg16_2 = _make_seg16(n_dirs=1, n_qoff=1)
_seg16_4 = _make_seg16(n_dirs=1, n_qoff=2)
_seg16_msg = _make_seg16(n_dirs=2, n_qoff=2)


def _deg_body(node, edge, dout, bout, accd, accb, zbuf, ones, ni, ei):
    c = lax.axis_index("c")
    s = lax.axis_index("s")
    ept = E // NC // NS
    _zero_vec_flat(zbuf, SPT)
    def orow(i, carry):
        ones[pl.ds(i * 16, 16)] = jnp.ones((16,), jnp.float32)
        return carry
    lax.fori_loop(0, (K + 16) // 16, orow, 0)
    sbase = s * SPT
    pltpu.sync_copy(zbuf, accd.at[pl.ds(sbase, SPT)])
    pltpu.sync_copy(zbuf, accb.at[pl.ds(sbase, SPT)])
    plsc.subcore_barrier()
    ebase = (c * NS + s) * ept

    def step(kk, carry):
        pltpu.sync_copy(node.at[pl.ds(ebase + kk * K, K)], ni)
        pltpu.sync_copy(edge.at[pl.ds(ebase + kk * K, K)], ei)
        pltpu.sync_copy(ones.at[pl.ds(0, K)], accd.at[ni], add=True)
        pltpu.sync_copy(ones.at[pl.ds(0, K)], accb.at[ei], add=True)
        return carry
    lax.fori_loop(0, ept // K, step, 0)
    plsc.subcore_barrier()
    pltpu.sync_copy(accd.at[pl.ds(sbase, SPT)], zbuf)
    pltpu.sync_copy(zbuf, dout.at[pl.ds(c * NPAD + sbase, SPT)])
    pltpu.sync_copy(accb.at[pl.ds(sbase, SPT)], zbuf)
    pltpu.sync_copy(zbuf, bout.at[pl.ds(c * NPAD + sbase, SPT)])


_seg_deg = pl.kernel(
    _deg_body,
    out_type=(jax.ShapeDtypeStruct((2 * NPAD,), jnp.float32),
              jax.ShapeDtypeStruct((2 * NPAD,), jnp.float32)),
    mesh=_MESH,
    compiler_params=_SC_PARAMS,
    scratch_types=[
        pltpu.VMEM_SHARED((NPAD,), jnp.float32),
        pltpu.VMEM_SHARED((NPAD,), jnp.float32),
        pltpu.VMEM((SPT,), jnp.float32),
        pltpu.VMEM((K + 16,), jnp.float32),
        pltpu.VMEM((K,), jnp.int32),
        pltpu.VMEM((K,), jnp.int32),
    ],
)


def _wsum_body(dinv, node, edge, out, acc, zbuf, ni, ei, rows, sem):
    c = lax.axis_index("c")
    s = lax.axis_index("s")
    ept = E // NC // NS
    _zero_vec_flat(zbuf, SPT)
    sbase = s * SPT
    pltpu.sync_copy(zbuf, acc.at[pl.ds(sbase, SPT)])
    plsc.subcore_barrier()
    ebase = (c * NS + s) * ept

    def step(kk, carry):
        pltpu.sync_copy(node.at[pl.ds(ebase + kk * K, K)], ni)
        pltpu.sync_copy(edge.at[pl.ds(ebase + kk * K, K)], ei)
        pltpu.async_copy(dinv.at[ni], rows, sem).wait()
        pltpu.sync_copy(rows, acc.at[ei], add=True)
        return carry
    lax.fori_loop(0, ept // K, step, 0)
    plsc.subcore_barrier()
    pltpu.sync_copy(acc.at[pl.ds(sbase, SPT)], zbuf)
    pltpu.sync_copy(zbuf, out.at[pl.ds(c * NPAD + sbase, SPT)])


_seg_wsum = pl.kernel(
    _wsum_body,
    out_type=jax.ShapeDtypeStruct((2 * NPAD,), jnp.float32),
    mesh=_MESH,
    compiler_params=_SC_PARAMS,
    scratch_types=[
        pltpu.VMEM_SHARED((NPAD,), jnp.float32),
        pltpu.VMEM((SPT,), jnp.float32),
        pltpu.VMEM((K,), jnp.int32),
        pltpu.VMEM((K,), jnp.int32),
        pltpu.VMEM((K,), jnp.float32),
        pltpu.SemaphoreType.DMA,
    ],
)


def _dotT(a, w):
    return lax.dot_general(a, w, (((1,), (1,)), ((), ())),
                           preferred_element_type=jnp.float32)


def _prep_body(xp, xl, w1, wt1, xw1, xe, mu1):
    v1 = _dotT(xp[...], w1[...])
    xw1[0, :, :] = v1[:, :FH]
    xw1[1, :, :] = v1[:, FH:]
    v = _dotT(xl[...], wt1[...])
    xe[...] = v
    m = jnp.maximum(v, 0.0)
    for q in range(4):
        mu1[q, :, :] = m[:, q * FH:(q + 1) * FH]


def _ef1_body(ep, bp, ef1, binv):
    b = bp[:, 0:1] + bp[:, 1:2]
    bi = jnp.where(b > 0, 1.0 / b, 0.0)
    binv[...] = bi
    ef1[0, :, :] = bi * ep[0]
    ef1[1, :, :] = bi * ep[1]


def _conv2_body(op, dp, b1, w2, xw2, dinv):
    d = dp[:, 0:1] + dp[:, 1:2]
    di = jnp.where(d > 0, 1.0 / d, 0.0)
    dinv[...] = di
    o = jnp.concatenate([op[0], op[1]], axis=1)
    z = jnp.maximum(di * o + b1[...], 0.0)
    v = _dotT(z, w2[...])
    for q in range(4):
        xw2[q, :, :] = v[:, q * FH:(q + 1) * FH]


def _hh_body(eq, wp, binv, acc):
    i = pl.program_id(0)
    v = (wp[:, 0:1] + wp[:, 1:2]) * binv[...]
    ef2 = jnp.concatenate([eq[0], eq[1], eq[2], eq[3]], axis=1)
    part = lax.dot_general(v, ef2, (((0,), (0,)), ((), ())),
                           preferred_element_type=jnp.float32)
    @pl.when(i == 0)
    def _():
        acc[...] = jnp.zeros_like(acc)
    acc[...] += part


def _s2v_mu(xe, mi, mo, wt2, wt3):
    m_in = jnp.concatenate([mi[0], mi[1], mi[2], mi[3]], axis=1)
    m_out = jnp.concatenate([mo[0], mo[1], mo[2], mo[3]], axis=1)
    return jnp.maximum(
        xe[...] + _dotT(m_in, wt2[...]) + _dotT(m_out, wt3[...]), 0.0)


def _s2v_body(xe, mi, mo, wt2, wt3, mu):
    m = _s2v_mu(xe, mi, mo, wt2, wt3)
    for q in range(4):
        mu[q, :, :] = m[:, q * FH:(q + 1) * FH]


def _s2v_final_body(xe, mi, mo, wt2, wt3, acc):
    i = pl.program_id(0)
    m = _s2v_mu(xe, mi, mo, wt2, wt3)
    part = jnp.sum(m, axis=0, keepdims=True)
    @pl.when(i == 0)
    def _():
        acc[...] = jnp.zeros_like(acc)
    acc[...] += part


def _fuse_body(hh, hg, b2, wf, bf, out):
    h_h = hh[...] * (1.0 / N) + b2[...]
    ht = jnp.concatenate([h_h, hg[...]], axis=1)
    out[...] = jnp.maximum(_dotT(ht, wf[...]) + bf[...], 0.0)


_GRID = N // RB


def _rows(shape):
    return pl.BlockSpec(shape, lambda i: (i,) + (0,) * (len(shape) - 1))


def _slabs(shape):
    return pl.BlockSpec(shape, lambda i: (0, i, 0))


def _full(shape):
    return pl.BlockSpec(shape, lambda i: (0,) * len(shape))


def _f32(shape):
    return jax.ShapeDtypeStruct(shape, jnp.float32)


_prep = pl.pallas_call(
    _prep_body, grid=(_GRID,),
    in_specs=[_rows((RB, 4)), _rows((RB, 5)), _full((32, 4)), _full((64, 5))],
    out_specs=[_slabs((2, RB, FH)), _rows((RB, 64)), _slabs((4, RB, FH))],
    out_shape=[_f32((2, NR, FH)), _f32((N, 64)), _f32((4, NR, FH))],
)

_ef1 = pl.pallas_call(
    _ef1_body, grid=(_GRID,),
    in_specs=[_slabs((2, RB, FH)), _rows((RB, 2))],
    out_specs=[_slabs((2, RB, FH)), _rows((RB, 1))],
    out_shape=[_f32((2, NR, FH)), _f32((N, 1))],
)

_conv2 = pl.pallas_call(
    _conv2_body, grid=(_GRID,),
    in_specs=[_slabs((2, RB, FH)), _rows((RB, 2)), _full((1, 32)),
              _full((64, 32))],
    out_specs=[_slabs((4, RB, FH)), _rows((RB, 1))],
    out_shape=[_f32((4, NR, FH)), _f32((N, 1))],
)

_hh = pl.pallas_call(
    _hh_body, grid=(_GRID,),
    in_specs=[_slabs((4, RB, FH)), _rows((RB, 2)), _rows((RB, 1))],
    out_specs=_full((1, 64)),
    out_shape=_f32((1, 64)),
)

_s2v_specs = [_rows((RB, 64)), _slabs((4, RB, FH)), _slabs((4, RB, FH)),
              _full((64, 64)), _full((64, 64))]

_s2v = pl.pallas_call(
    _s2v_body, grid=(_GRID,),
    in_specs=_s2v_specs,
    out_specs=_slabs((4, RB, FH)),
    out_shape=_f32((4, NR, FH)),
)

_s2v_final = pl.pallas_call(
    _s2v_final_body, grid=(_GRID,),
    in_specs=_s2v_specs,
    out_specs=_full((1, 64)),
    out_shape=_f32((1, 64)),
)

_fuse = pl.pallas_call(
    _fuse_body, grid=(1,),
    in_specs=[_full((1, 64)), _full((1, 64)), _full((1, 64)),
              _full((128, 128)), _full((1, 128))],
    out_specs=_full((1, 128)),
    out_shape=_f32((1, 128)),
)


def kernel(x_phy, hyperedge_index, x_log, edge_index, W_hg1, b_hg1, W_hg2,
           b_hg2, W_t1, W_t2, W_t3, W_fuse, b_fuse):
    node, edge = hyperedge_index[0], hyperedge_index[1]
    src, dst = edge_index[0], edge_index[1]
    node2 = jnp.concatenate([node, node + NR])
    edge2 = jnp.concatenate([edge, edge + NR])
    node4 = jnp.concatenate([node, node + NR, node + 2 * NR, node + 3 * NR])
    src4 = jnp.concatenate([src, src + NR, src + 2 * NR, src + 3 * NR])
    dst4 = jnp.concatenate([dst, dst + NR, dst + 2 * NR, dst + 3 * NR])

    xw1, xe, mu = _prep(x_phy, x_log, W_hg1, W_t1)

    dp_flat, bp_flat = _seg_deg(node, edge)
    dp = dp_flat.reshape(2, NPAD).T
    bp = bp_flat.reshape(2, NPAD).T

    ef1_p = _seg16_2(xw1.reshape(2 * NR, FH), node2, edge).reshape(2, NR, FH)
    ef1, binv = _ef1(ef1_p, bp)
    out1_p = _seg16_2(ef1.reshape(2 * NR, FH), edge2, node).reshape(2, NR, FH)
    xw2, dinv = _conv2(out1_p, dp, b_hg1.reshape(1, 32), W_hg2)
    wsum_flat = _seg_wsum(dinv.reshape(N), node, edge)
    wp = wsum_flat.reshape(2, NPAD).T
    ef2_p = _seg16_4(xw2.reshape(4 * NR, FH), node4, edge).reshape(4, NR, FH)
    hh = _hh(ef2_p, wp, binv)

    for it in range(2):
        flat = mu.reshape(4 * NR, FH)
        mi_p = _seg16_4(flat, src4, dst).reshape(4, NR, FH)
        mo_p = _seg16_4(flat, dst4, src).reshape(4, NR, FH)
        if it == 0:
            mu = _s2v(xe, mi_p, mo_p, W_t2, W_t3)
        else:
            hg = _s2v_final(xe, mi_p, mo_p, W_t2, W_t3)

    out = _fuse(hh, hg, b_hg2.reshape(1, 64), W_fuse, b_fuse.reshape(1, 128))
    return out.reshape(128)
